# baseline (speedup 1.0000x reference)
import functools

import jax
import jax.numpy as jnp
from jax import lax
from jax.experimental import pallas as pl
from jax.experimental.pallas import tpu as pltpu
from jax.experimental.pallas import tpu_sc as plsc

_BATCH = 1024
_SEQ = 200
_N = _BATCH * _SEQ
_NC = 2
_NS = 16
_NW = _NC * _NS
_PER_W = _N // _NW
_CHUNK = 128
_NCHUNK = _PER_W // _CHUNK
_HVEC = _PER_W // 16
_EDIM = 64
_PDIM = 128
_MDIM = 512
_MULT_A = 36313
_MULT_B = 27191
_MOD = 999999


def _hash_body(tok_hbm, idx_hbm, tok_v, idx_v):
    wid = lax.axis_index("s") * _NC + lax.axis_index("c")
    nrow = _N // _PDIM // _NW

    def tok_row(r, _):
        pltpu.sync_copy(tok_hbm.at[wid * nrow + r],
                        tok_v.at[pl.ds(8 + r * _PDIM, _PDIM)])
        return 0

    lax.fori_loop(0, nrow, tok_row, 0)

    def hash_step(k, _):
        cur = tok_v[pl.ds(8 + k * 16, 16)]
        prev = tok_v[pl.ds(7 + k * 16, 16)]
        h = (_MULT_A * cur ^ _MULT_B * prev) % _MOD
        pos = k * 16 + lax.iota(jnp.int32, 16)
        idx_v[pl.ds(k * 16, 16)] = jnp.where(pos % _SEQ == 0, _MOD, h)
        return 0

    lax.fori_loop(0, nrow * _PDIM // 16, hash_step, 0)

    def idx_row(r, _):
        pltpu.sync_copy(idx_v.at[pl.ds(r * _PDIM, _PDIM)],
                        idx_hbm.at[wid * nrow + r])
        return 0

    lax.fori_loop(0, nrow, idx_row, 0)


_GC = 64


def _gather_body(nchunk, idx_hbm, table_hbm, out_hbm, idx_v, rows_v, sem):
    wid = lax.axis_index("s") * _NC + lax.axis_index("c")

    def gather_step(ci, _):
        c = ci * _NW + wid
        pltpu.sync_copy(idx_hbm.at[c // 2, pl.ds((c % 2) * _GC, _GC)],
                        idx_v)
        pltpu.async_copy(table_hbm.at[idx_v], rows_v, sem).wait()
        pltpu.sync_copy(
            rows_v,
            out_hbm.at[pl.ds(c * _GC, _GC), pl.ds(0, _EDIM)])
        return 0

    lax.fori_loop(0, nchunk, gather_step, 0)


_MESHA = dict(core_axis_name="c", subcore_axis_name="s", num_cores=_NC,
              num_subcores=_NS)


def _make_sc_hash():
    return pl.kernel(
        _hash_body,
        out_type=jax.ShapeDtypeStruct((_N // _PDIM, _PDIM), jnp.int32),
        mesh=plsc.VectorSubcoreMesh(**_MESHA),
        scratch_types=[
            pltpu.VMEM((_PER_W + 8,), jnp.int32),
            pltpu.VMEM((_PER_W,), jnp.int32),
        ],
        compiler_params=pltpu.CompilerParams(use_tc_tiling_on_sc=False),
    )


def _make_sc_gather(nrows):
    nchunk = nrows // (_NW * _GC)
    assert nrows % (_NW * _GC) == 0
    return pl.kernel(
        functools.partial(_gather_body, nchunk),
        out_type=jax.ShapeDtypeStruct((nrows, _PDIM), jnp.float32),
        mesh=plsc.VectorSubcoreMesh(**_MESHA),
        scratch_types=[
            pltpu.VMEM((_GC,), jnp.int32),
            pltpu.VMEM((_GC, _EDIM), jnp.float32),
            pltpu.SemaphoreType.DMA,
        ],
        compiler_params=pltpu.CompilerParams(use_tc_tiling_on_sc=False),
    )


_RB = 6400


def _mm_body(h_ref, w_ref, o_ref):
    o_ref[...] = jnp.dot(h_ref[:, :_EDIM], w_ref[...],
                         preferred_element_type=jnp.float32)


def _mm_body_acc(h_ref, w_ref, acc_ref, o_ref):
    del acc_ref
    o_ref[...] = jnp.dot(h_ref[:, :_EDIM], w_ref[...],
                         preferred_element_type=jnp.float32)


def _tc_project_part(h, w, acc, part):
    n = h.shape[0]
    off = part * (n // _RB)
    specs = [
        pl.BlockSpec((_RB, _PDIM), lambda i: (i, 0)),
        pl.BlockSpec((_EDIM, _MDIM), lambda i: (0, 0)),
    ]
    args = [h, w]
    body = _mm_body
    aliases = {}
    if part:
        specs.append(pl.BlockSpec(memory_space=pl.MemorySpace.ANY))
        args.append(acc)
        body = _mm_body_acc
        aliases = {2: 0}
    return pl.pallas_call(
        body,
        grid=(n // _RB,),
        in_specs=specs,
        out_specs=pl.BlockSpec((_RB, _MDIM), lambda i: (off + i, 0)),
        out_shape=jax.ShapeDtypeStruct((_N, _MDIM), jnp.float32),
        input_output_aliases=aliases,
    )(*args)


_K = 4


@jax.jit
def _pipeline(token_ids, embed_weight, proj_weight, scale):
    ii = jnp.arange(_N, dtype=jnp.int32).reshape(_N // _PDIM, _PDIM)
    tok128 = token_ids[ii // _SEQ, ii % _SEQ]
    w = (proj_weight * scale).T
    idx = _make_sc_hash()(tok128)
    nrows = _N // _K
    sc = _make_sc_gather(nrows)
    trows = nrows // _PDIM
    gs = [sc(lax.slice_in_dim(idx, k * trows, (k + 1) * trows),
             embed_weight) for k in range(_K)]
    acc = None
    for k in range(_K):
        acc = _tc_project_part(gs[k], w, acc, k)
    return acc.reshape(_BATCH, _SEQ, _MDIM)


def kernel(token_ids, embed_weight, proj_weight, scale):
    return _pipeline(token_ids, embed_weight, proj_weight, scale)


# speedup vs baseline: 1.0331x; 1.0331x over previous
import functools

import jax
import jax.numpy as jnp
from jax import lax
from jax.experimental import pallas as pl
from jax.experimental.pallas import tpu as pltpu
from jax.experimental.pallas import tpu_sc as plsc

_BATCH = 1024
_SEQ = 200
_N = _BATCH * _SEQ
_NC = 2
_NS = 16
_NW = _NC * _NS
_PER_W = _N // _NW
_CHUNK = 128
_NCHUNK = _PER_W // _CHUNK
_HVEC = _PER_W // 16
_EDIM = 64
_PDIM = 128
_MDIM = 512
_MULT_A = 36313
_MULT_B = 27191
_MOD = 999999


def _hash_body(tok_hbm, idx_hbm, tok_v, idx_v):
    wid = lax.axis_index("s") * _NC + lax.axis_index("c")
    nrow = _N // _PDIM // _NW

    def tok_row(r, _):
        pltpu.sync_copy(tok_hbm.at[wid * nrow + r],
                        tok_v.at[pl.ds(8 + r * _PDIM, _PDIM)])
        return 0

    lax.fori_loop(0, nrow, tok_row, 0)

    def hash_step(k, _):
        cur = tok_v[pl.ds(8 + k * 16, 16)]
        prev = tok_v[pl.ds(7 + k * 16, 16)]
        h = (_MULT_A * cur ^ _MULT_B * prev) % _MOD
        pos = k * 16 + lax.iota(jnp.int32, 16)
        idx_v[pl.ds(k * 16, 16)] = jnp.where(pos % _SEQ == 0, _MOD, h)
        return 0

    lax.fori_loop(0, nrow * _PDIM // 16, hash_step, 0)

    def idx_row(r, _):
        pltpu.sync_copy(idx_v.at[pl.ds(r * _PDIM, _PDIM)],
                        idx_hbm.at[wid * nrow + r])
        return 0

    lax.fori_loop(0, nrow, idx_row, 0)


_GC = 128


def _gather_body(nchunk, idx_hbm, table_hbm, out_hbm, idx_v, rows_v, sem):
    wid = lax.axis_index("s") * _NC + lax.axis_index("c")

    def gather_step(ci, _):
        c = ci * _NW + wid
        pltpu.sync_copy(idx_hbm.at[c], idx_v)
        pltpu.async_copy(table_hbm.at[idx_v], rows_v, sem).wait()
        pltpu.sync_copy(
            rows_v,
            out_hbm.at[pl.ds(c * _GC, _GC), pl.ds(0, _EDIM)])
        return 0

    lax.fori_loop(0, nchunk, gather_step, 0)


_MESHA = dict(core_axis_name="c", subcore_axis_name="s", num_cores=_NC,
              num_subcores=_NS)


def _make_sc_hash():
    return pl.kernel(
        _hash_body,
        out_type=jax.ShapeDtypeStruct((_N // _PDIM, _PDIM), jnp.int32),
        mesh=plsc.VectorSubcoreMesh(**_MESHA),
        scratch_types=[
            pltpu.VMEM((_PER_W + 8,), jnp.int32),
            pltpu.VMEM((_PER_W,), jnp.int32),
        ],
        compiler_params=pltpu.CompilerParams(use_tc_tiling_on_sc=False),
    )


def _make_sc_gather(nrows):
    nchunk = nrows // (_NW * _GC)
    assert nrows % (_NW * _GC) == 0
    return pl.kernel(
        functools.partial(_gather_body, nchunk),
        out_type=jax.ShapeDtypeStruct((nrows, _PDIM), jnp.float32),
        mesh=plsc.VectorSubcoreMesh(**_MESHA),
        scratch_types=[
            pltpu.VMEM((_GC,), jnp.int32),
            pltpu.VMEM((_GC, _EDIM), jnp.float32),
            pltpu.SemaphoreType.DMA,
        ],
        compiler_params=pltpu.CompilerParams(use_tc_tiling_on_sc=False),
    )


_RB = 8192


def _mm_body(h_ref, w_ref, o_ref):
    o_ref[...] = jnp.dot(h_ref[:, :_EDIM], w_ref[...],
                         preferred_element_type=jnp.float32)


def _mm_body_acc(h_ref, w_ref, acc_ref, o_ref):
    del acc_ref
    o_ref[...] = jnp.dot(h_ref[:, :_EDIM], w_ref[...],
                         preferred_element_type=jnp.float32)


def _tc_project_part(h, w, acc, part):
    n = h.shape[0]
    off = part * (n // _RB)
    specs = [
        pl.BlockSpec((_RB, _PDIM), lambda i: (i, 0)),
        pl.BlockSpec((_EDIM, _MDIM), lambda i: (0, 0)),
    ]
    args = [h, w]
    body = _mm_body
    aliases = {}
    if part:
        specs.append(pl.BlockSpec(memory_space=pl.MemorySpace.ANY))
        args.append(acc)
        body = _mm_body_acc
        aliases = {2: 0}
    return pl.pallas_call(
        body,
        grid=(n // _RB,),
        in_specs=specs,
        out_specs=pl.BlockSpec((_RB, _MDIM), lambda i: (off + i, 0)),
        out_shape=jax.ShapeDtypeStruct((_N, _MDIM), jnp.float32),
        input_output_aliases=aliases,
    )(*args)


_K = 2


@jax.jit
def _pipeline(token_ids, embed_weight, proj_weight, scale):
    ii = jnp.arange(_N, dtype=jnp.int32).reshape(_N // _PDIM, _PDIM)
    tok128 = token_ids[ii // _SEQ, ii % _SEQ]
    w = (proj_weight * scale).T
    idx = _make_sc_hash()(tok128)
    nrows = _N // _K
    sc = _make_sc_gather(nrows)
    trows = nrows // _PDIM
    gs = [sc(lax.slice_in_dim(idx, k * trows, (k + 1) * trows),
             embed_weight) for k in range(_K)]
    acc = None
    for k in range(_K):
        acc = _tc_project_part(gs[k], w, acc, k)
    return acc.reshape(_BATCH, _SEQ, _MDIM)


def kernel(token_ids, embed_weight, proj_weight, scale):
    return _pipeline(token_ids, embed_weight, proj_weight, scale)
